# baseline (speedup 1.0000x reference)
import functools

import jax
import jax.numpy as jnp
from jax import lax
from jax.experimental import pallas as pl
from jax.experimental.pallas import tpu as pltpu
from jax.experimental.pallas import tpu_sc as plsc

B = 4096
T = 200
D = 64
NW = 32
BPW = B // NW
HALF = T // 2
L = 16

_TSTARTS = tuple(range(0, T - L + 1, L)) + (T - L,)


def _body(rid_hbm, pos_hbm, table_hbm, out_hbm, idx_v, rows_v, obuf_v, pos_v,
          sem):
    wid = lax.axis_index("s") * 2 + lax.axis_index("c")
    base_b = wid * BPW

    pltpu.sync_copy(pos_hbm, pos_v)
    pltpu.sync_copy(rid_hbm.at[pl.ds(base_b, BPW)], idx_v)

    iota = lax.iota(jnp.int32, L)

    def per_batch(i, carry):
        gb = base_b + i
        cp0 = pltpu.async_copy(table_hbm.at[idx_v.at[i, 0]],
                               rows_v.at[pl.ds(0, HALF)], sem)
        cp1 = pltpu.async_copy(table_hbm.at[idx_v.at[i, 1]],
                               rows_v.at[pl.ds(HALF, HALF)], sem)
        cp0.wait()
        cp1.wait()

        def per_d(d, carry2):
            dvec = jnp.zeros((L,), dtype=jnp.int32) + d
            off = d * T
            for ts in _TSTARTS:
                vals = plsc.load_gather(rows_v, [iota + ts, dvec])
                pv = pos_v[pl.ds(off + ts, L)]
                obuf_v[pl.ds(off + ts, L)] = vals + pv
            return carry2

        lax.fori_loop(0, D, per_d, 0, unroll=False)
        pltpu.sync_copy(obuf_v, out_hbm.at[gb])
        return carry

    lax.fori_loop(0, BPW, per_batch, 0, unroll=False)


def kernel(rid, table, pos):
    rid3 = rid.astype(jnp.int32).reshape(B, 2, HALF)
    posf = pos.astype(jnp.float32).reshape(D * T)

    mesh = plsc.VectorSubcoreMesh(core_axis_name="c", subcore_axis_name="s")
    run = functools.partial(
        pl.kernel,
        mesh=mesh,
        out_type=jax.ShapeDtypeStruct((B, D * T), jnp.float32),
        scratch_types=[
            pltpu.VMEM((BPW, 2, HALF), jnp.int32),
            pltpu.VMEM((T, D), jnp.float32),
            pltpu.VMEM((D * T,), jnp.float32),
            pltpu.VMEM((D * T,), jnp.float32),
            pltpu.SemaphoreType.DMA,
        ],
        compiler_params=pltpu.CompilerParams(needs_layout_passes=False,
                                             use_tc_tiling_on_sc=False),
    )(_body)
    out = run(rid3, posf, table)
    return out.reshape(B, D, T)


# speedup vs baseline: 1.0982x; 1.0982x over previous
import functools

import jax
import jax.numpy as jnp
from jax import lax
from jax.experimental import pallas as pl
from jax.experimental.pallas import tpu as pltpu
from jax.experimental.pallas import tpu_sc as plsc

B = 4096
T = 200
D = 64
NW = 32
BPW = B // NW
HALF = T // 2
L = 16

_TSTARTS = tuple(range(0, T - L + 1, L)) + (T - L,)


def _body(rid_hbm, pos_hbm, table_hbm, out_hbm,
          idx_v, pos_v, rows0, rows1, obuf0, obuf1,
          sg0, sg1, so0, so1):
    wid = lax.axis_index("s") * 2 + lax.axis_index("c")
    base_b = wid * BPW
    rows = (rows0, rows1)
    obuf = (obuf0, obuf1)
    sg = (sg0, sg1)
    so = (so0, so1)

    pltpu.sync_copy(pos_hbm, pos_v)
    pltpu.sync_copy(rid_hbm.at[pl.ds(base_b, BPW)], idx_v)

    iota = lax.iota(jnp.int32, L)

    def issue_gather(i, k):
        pltpu.async_copy(table_hbm.at[idx_v.at[i, 0]],
                         rows[k].at[pl.ds(0, HALF)], sg[k])
        pltpu.async_copy(table_hbm.at[idx_v.at[i, 1]],
                         rows[k].at[pl.ds(HALF, HALF)], sg[k])

    def wait_gather(k):
        pltpu.make_async_copy(table_hbm.at[idx_v.at[0, 0]],
                              rows[k].at[pl.ds(0, HALF)], sg[k]).wait()
        pltpu.make_async_copy(table_hbm.at[idx_v.at[0, 1]],
                              rows[k].at[pl.ds(HALF, HALF)], sg[k]).wait()

    def wait_out(k):
        pltpu.make_async_copy(obuf[k], out_hbm.at[0], so[k]).wait()

    issue_gather(0, 0)
    issue_gather(1, 1)

    def loop_body(j, carry):
        for k in (0, 1):
            b = 2 * j + k
            wait_gather(k)

            @pl.when(b + 2 < BPW)
            def _():
                issue_gather(b + 2, k)

            @pl.when(b >= 2)
            def _():
                wait_out(k)

            def per_d(d, carry2):
                dvec = jnp.zeros((L,), dtype=jnp.int32) + d
                off = d * T
                for ts in _TSTARTS:
                    vals = plsc.load_gather(rows[k], [iota + ts, dvec])
                    pv = pos_v[pl.ds(off + ts, L)]
                    obuf[k][pl.ds(off + ts, L)] = vals + pv
                return carry2

            lax.fori_loop(0, D, per_d, 0, unroll=2)
            pltpu.async_copy(obuf[k], out_hbm.at[base_b + b], so[k])
        return carry

    lax.fori_loop(0, BPW // 2, loop_body, 0, unroll=False)
    wait_out(0)
    wait_out(1)


def kernel(rid, table, pos):
    rid3 = rid.astype(jnp.int32).reshape(B, 2, HALF)
    posf = pos.astype(jnp.float32).reshape(D * T)

    mesh = plsc.VectorSubcoreMesh(core_axis_name="c", subcore_axis_name="s")
    run = functools.partial(
        pl.kernel,
        mesh=mesh,
        out_type=jax.ShapeDtypeStruct((B, D * T), jnp.float32),
        scratch_types=[
            pltpu.VMEM((BPW, 2, HALF), jnp.int32),
            pltpu.VMEM((D * T,), jnp.float32),
            pltpu.VMEM((T, D), jnp.float32),
            pltpu.VMEM((T, D), jnp.float32),
            pltpu.VMEM((D * T,), jnp.float32),
            pltpu.VMEM((D * T,), jnp.float32),
            pltpu.SemaphoreType.DMA,
            pltpu.SemaphoreType.DMA,
            pltpu.SemaphoreType.DMA,
            pltpu.SemaphoreType.DMA,
        ],
        compiler_params=pltpu.CompilerParams(needs_layout_passes=False,
                                             use_tc_tiling_on_sc=False),
    )(_body)
    out = run(rid3, posf, table)
    return out.reshape(B, D, T)


# speedup vs baseline: 1.9104x; 1.7396x over previous
import functools

import jax
import jax.numpy as jnp
from jax import lax
from jax.experimental import pallas as pl
from jax.experimental.pallas import tpu as pltpu
from jax.experimental.pallas import tpu_sc as plsc

B = 4096
T = 200
D = 64
NW = 32
BPW = B // NW
HALF = T // 2
L = 16
OSTRIDE = 201


def _body(rid_hbm, post_hbm, table_hbm, out_hbm,
          idx_v, post_v, rows0, rows1, obuf0, obuf1,
          sg0, sg1, so0, so1):
    wid = lax.axis_index("s") * 2 + lax.axis_index("c")
    base_b = wid * BPW
    rows = (rows0, rows1)
    obuf = (obuf0, obuf1)
    sg = (sg0, sg1)
    so = (so0, so1)

    pltpu.sync_copy(post_hbm, post_v)
    pltpu.sync_copy(rid_hbm.at[pl.ds(base_b, BPW)], idx_v)

    iota = lax.iota(jnp.int32, L)

    def issue_gather(i, k):
        pltpu.async_copy(table_hbm.at[idx_v.at[i, 0]],
                         rows[k].at[pl.ds(0, HALF)], sg[k])
        pltpu.async_copy(table_hbm.at[idx_v.at[i, 1]],
                         rows[k].at[pl.ds(HALF, HALF)], sg[k])

    def wait_gather(k):
        pltpu.make_async_copy(table_hbm.at[idx_v.at[0, 0]],
                              rows[k].at[pl.ds(0, HALF)], sg[k]).wait()
        pltpu.make_async_copy(table_hbm.at[idx_v.at[0, 1]],
                              rows[k].at[pl.ds(HALF, HALF)], sg[k]).wait()

    def issue_out(b, k):
        pltpu.async_copy(obuf[k].at[:, pl.ds(0, T)], out_hbm.at[base_b + b],
                         so[k])

    def wait_out(k):
        pltpu.make_async_copy(obuf[k].at[:, pl.ds(0, T)], out_hbm.at[0],
                              so[k]).wait()

    issue_gather(0, 0)
    issue_gather(1, 1)

    def loop_body(j, carry):
        for k in (0, 1):
            b = 2 * j + k
            wait_gather(k)

            @pl.when(b >= 2)
            def _():
                wait_out(k)

            @plsc.parallel_loop(0, T, 1, unroll=4)
            def per_t(t):
                tvec = jnp.zeros((L,), dtype=jnp.int32) + t
                for dc in range(D // L):
                    v = rows[k][t, pl.ds(dc * L, L)]
                    p = post_v[t, pl.ds(dc * L, L)]
                    plsc.store_scatter(obuf[k], [dc * L + iota, tvec], v + p)

            issue_out(b, k)

            @pl.when(b + 2 < BPW)
            def _():
                issue_gather(b + 2, k)
        return carry

    lax.fori_loop(0, BPW // 2, loop_body, 0, unroll=False)
    wait_out(0)
    wait_out(1)


def kernel(rid, table, pos):
    rid3 = rid.astype(jnp.int32).reshape(B, 2, HALF)
    post = jnp.transpose(pos.astype(jnp.float32).reshape(D, T), (1, 0))

    mesh = plsc.VectorSubcoreMesh(core_axis_name="c", subcore_axis_name="s")
    run = functools.partial(
        pl.kernel,
        mesh=mesh,
        out_type=jax.ShapeDtypeStruct((B, D, T), jnp.float32),
        scratch_types=[
            pltpu.VMEM((BPW, 2, HALF), jnp.int32),
            pltpu.VMEM((T, D), jnp.float32),
            pltpu.VMEM((T, D), jnp.float32),
            pltpu.VMEM((T, D), jnp.float32),
            pltpu.VMEM((D, OSTRIDE), jnp.float32),
            pltpu.VMEM((D, OSTRIDE), jnp.float32),
            pltpu.SemaphoreType.DMA,
            pltpu.SemaphoreType.DMA,
            pltpu.SemaphoreType.DMA,
            pltpu.SemaphoreType.DMA,
        ],
        compiler_params=pltpu.CompilerParams(needs_layout_passes=False,
                                             use_tc_tiling_on_sc=False),
    )(_body)
    return run(rid3, post, table)


# speedup vs baseline: 3.3400x; 1.7483x over previous
import functools

import jax
import jax.numpy as jnp
from jax import lax
from jax.experimental import pallas as pl
from jax.experimental.pallas import tpu as pltpu
from jax.experimental.pallas import tpu_sc as plsc

B = 4096
T = 200
D = 64
NW = 32
L = 16
TJ = T // 8
BJ = B // 128
NCHUNK = T // 2
OSTRIDE = 257


def _body(rid_hbm, post_hbm, table_hbm, out_hbm,
          idx_v, post_v, rows0, rows1, obuf0, obuf1,
          sg0, sg1, so0, so1):
    w = lax.axis_index("s") * 2 + lax.axis_index("c")
    rows = (rows0, rows1)
    obuf = (obuf0, obuf1)
    sg = (sg0, sg1)
    so = (so0, so1)

    pltpu.sync_copy(post_hbm, post_v)
    pltpu.sync_copy(rid_hbm.at[:, w], idx_v)

    iota = lax.iota(jnp.int32, L)

    def tj_tt(c):
        return lax.shift_right_logical(c, 2), lax.mul(lax.rem(c, 4), 2)

    def issue_gather(c, k):
        tj, tt0 = tj_tt(c)
        for h in (0, 1):
            pltpu.async_copy(table_hbm.at[idx_v.at[tj, tt0 + h]],
                             rows[k].at[h], sg[k])

    def wait_gather(k):
        for h in (0, 1):
            pltpu.make_async_copy(table_hbm.at[idx_v.at[0, 0]],
                                  rows[k].at[h], sg[k]).wait()

    def issue_out(c, k):
        tj, tt0 = tj_tt(c)
        pltpu.async_copy(obuf[k].at[:, pl.ds(0, 256)],
                         out_hbm.at[:, tj, w, pl.ds(tt0 * 128, 256)], so[k])

    def wait_out(k):
        pltpu.make_async_copy(obuf[k].at[:, pl.ds(0, 256)],
                              out_hbm.at[:, 0, 0, pl.ds(0, 256)], so[k]).wait()

    issue_gather(0, 0)
    issue_gather(1, 1)

    def loop_body(j, carry):
        for k in (0, 1):
            c = 2 * j + k
            tj, tt0 = tj_tt(c)
            wait_gather(k)

            @pl.when(c >= 2)
            def _():
                wait_out(k)

            for h in (0, 1):
                t = lax.add(lax.add(lax.mul(tj, 8), tt0), h)
                for dc in range(D // L):
                    dvec = dc * L + iota
                    pvec = post_v[t, pl.ds(dc * L, L)]
                    col0 = h * 128

                    @plsc.parallel_loop(0, 128, 1, unroll=8)
                    def per_bb(bb):
                        v = rows[k][h, bb, pl.ds(dc * L, L)]
                        plsc.store_scatter(
                            obuf[k], [dvec, jnp.zeros((L,), jnp.int32)
                                      + (col0 + bb)], v + pvec)

            issue_out(c, k)

            @pl.when(c + 2 < NCHUNK)
            def _():
                issue_gather(c + 2, k)
        return carry

    lax.fori_loop(0, NCHUNK // 2, loop_body, 0, unroll=False)
    wait_out(0)
    wait_out(1)


def kernel(rid, table, pos):
    rid4 = (rid.astype(jnp.int32)
            .reshape(BJ, 128, TJ, 8).transpose(2, 0, 3, 1))
    post = jnp.transpose(pos.astype(jnp.float32).reshape(D, T), (1, 0))

    mesh = plsc.VectorSubcoreMesh(core_axis_name="c", subcore_axis_name="s")
    run = functools.partial(
        pl.kernel,
        mesh=mesh,
        out_type=jax.ShapeDtypeStruct((D, TJ, BJ, 1024), jnp.float32),
        scratch_types=[
            pltpu.VMEM((TJ, 8, 128), jnp.int32),
            pltpu.VMEM((T, D), jnp.float32),
            pltpu.VMEM((2, 128, D), jnp.float32),
            pltpu.VMEM((2, 128, D), jnp.float32),
            pltpu.VMEM((D, OSTRIDE), jnp.float32),
            pltpu.VMEM((D, OSTRIDE), jnp.float32),
            pltpu.SemaphoreType.DMA,
            pltpu.SemaphoreType.DMA,
            pltpu.SemaphoreType.DMA,
            pltpu.SemaphoreType.DMA,
        ],
        compiler_params=pltpu.CompilerParams(needs_layout_passes=False,
                                             use_tc_tiling_on_sc=False),
    )(_body)
    out5 = run(rid4, post, table)
    return (out5.reshape(D, TJ, BJ, 8, 128)
            .transpose(2, 4, 0, 1, 3)
            .reshape(B, D, T))
